# baseline (speedup 1.0000x reference)
import jax
import jax.numpy as jnp
from jax.experimental import pallas as pl
from jax.experimental.pallas import tpu as pltpu


def _copy_body(e_hbm, e_out, ev, s1, s2):
    c = pltpu.make_async_copy(e_hbm.at[pl.ds(0, 8), :], ev, s1)
    c.start()
    c.wait()
    o = pltpu.make_async_copy(ev, e_out.at[pl.ds(0, 8), :], s2)
    o.start()
    o.wait()


def kernel(x, x_lstm, encoded_z_gnss, edge_index, edge_attr):
    n_edges, d_edge = edge_attr.shape
    e_view = edge_attr.reshape((n_edges * d_edge) // 128, 128)
    e_out = pl.pallas_call(
        _copy_body,
        out_shape=jax.ShapeDtypeStruct(e_view.shape, e_view.dtype),
        in_specs=[pl.BlockSpec(memory_space=pl.ANY)],
        out_specs=pl.BlockSpec(memory_space=pl.ANY),
        scratch_shapes=[
            pltpu.MemorySpace.VMEM((8, 128), jnp.float32),
            pltpu.SemaphoreType.DMA,
            pltpu.SemaphoreType.DMA,
        ],
    )(e_view)
    return (x, e_out.reshape(n_edges, d_edge))


# speedup vs baseline: 1.1574x; 1.1574x over previous
import jax
import jax.numpy as jnp
from jax.experimental import pallas as pl
from jax.experimental.pallas import tpu as pltpu

_CX = 10
_CE = 10
_EBUF = 2


def _copy_body(x_hbm, e_hbm, x_out, e_out, x_v, e_v0, e_v1,
               xin_sem, xout_sem, ein_sem, eout_sem):
    nx = x_hbm.shape[0] // _CX
    ne = e_hbm.shape[0] // _CE
    ebufs = (e_v0, e_v1)

    x_loads = []
    for i in range(_CX):
        c = pltpu.make_async_copy(
            x_hbm.at[pl.ds(i * nx, nx), :], x_v.at[pl.ds(i * nx, nx), :],
            xin_sem.at[i])
        c.start()
        x_loads.append(c)

    e_loads = {}
    e_stores = {}
    for i in range(min(_EBUF, _CE)):
        c = pltpu.make_async_copy(
            e_hbm.at[pl.ds(i * ne, ne), :], ebufs[i % _EBUF],
            ein_sem.at[i % _EBUF])
        c.start()
        e_loads[i] = c

    x_stores = []
    for i in range(_CE):
        e_loads[i].wait()
        s = pltpu.make_async_copy(
            ebufs[i % _EBUF], e_out.at[pl.ds(i * ne, ne), :],
            eout_sem.at[i % _EBUF])
        s.start()
        e_stores[i] = s
        nxt = i + _EBUF
        if nxt < _CE:
            e_stores[i].wait()
            c = pltpu.make_async_copy(
                e_hbm.at[pl.ds(nxt * ne, ne), :], ebufs[nxt % _EBUF],
                ein_sem.at[nxt % _EBUF])
            c.start()
            e_loads[nxt] = c
        if i < _CX:
            x_loads[i].wait()
            s = pltpu.make_async_copy(
                x_v.at[pl.ds(i * nx, nx), :], x_out.at[pl.ds(i * nx, nx), :],
                xout_sem.at[i])
            s.start()
            x_stores.append(s)

    for s in x_stores:
        s.wait()
    for i in range(_CE - _EBUF, _CE):
        e_stores[i].wait()


def kernel(x, x_lstm, encoded_z_gnss, edge_index, edge_attr):
    n_nodes, d_feat = x.shape
    n_edges, d_edge = edge_attr.shape
    x_out, e_out = pl.pallas_call(
        _copy_body,
        out_shape=(
            jax.ShapeDtypeStruct(x.shape, x.dtype),
            jax.ShapeDtypeStruct(edge_attr.shape, edge_attr.dtype),
        ),
        in_specs=[
            pl.BlockSpec(memory_space=pl.ANY),
            pl.BlockSpec(memory_space=pl.ANY),
        ],
        out_specs=(
            pl.BlockSpec(memory_space=pl.ANY),
            pl.BlockSpec(memory_space=pl.ANY),
        ),
        scratch_shapes=[
            pltpu.MemorySpace.VMEM((n_nodes, d_feat), jnp.float32),
            pltpu.MemorySpace.VMEM((n_edges // _CE, d_edge), jnp.float32),
            pltpu.MemorySpace.VMEM((n_edges // _CE, d_edge), jnp.float32),
            pltpu.SemaphoreType.DMA((_CX,)),
            pltpu.SemaphoreType.DMA((_CX,)),
            pltpu.SemaphoreType.DMA((_EBUF,)),
            pltpu.SemaphoreType.DMA((_EBUF,)),
        ],
    )(x, edge_attr)
    return (x_out, e_out)


# speedup vs baseline: 1.1703x; 1.0111x over previous
import jax
import jax.numpy as jnp
from jax.experimental import pallas as pl
from jax.experimental.pallas import tpu as pltpu

_CX = 10
_CE = 20
_EBUF = 6


def _copy_body(x_hbm, e_hbm, x_out, e_out, x_v, *rest):
    ebufs = rest[:_EBUF]
    xin_sem, xout_sem, ein_sem, eout_sem = rest[_EBUF:]
    nx = x_hbm.shape[0] // _CX
    ne = e_hbm.shape[0] // _CE

    x_loads = []
    for i in range(_CX):
        c = pltpu.make_async_copy(
            x_hbm.at[pl.ds(i * nx, nx), :], x_v.at[pl.ds(i * nx, nx), :],
            xin_sem.at[i])
        c.start()
        x_loads.append(c)

    e_loads = {}
    e_stores = {}
    for i in range(min(_EBUF, _CE)):
        c = pltpu.make_async_copy(
            e_hbm.at[pl.ds(i * ne, ne), :], ebufs[i % _EBUF],
            ein_sem.at[i % _EBUF])
        c.start()
        e_loads[i] = c

    x_stores = []
    for i in range(_CX):
        x_loads[i].wait()
        s = pltpu.make_async_copy(
            x_v.at[pl.ds(i * nx, nx), :], x_out.at[pl.ds(i * nx, nx), :],
            xout_sem.at[i])
        s.start()
        x_stores.append(s)

    for i in range(_CE):
        e_loads[i].wait()
        s = pltpu.make_async_copy(
            ebufs[i % _EBUF], e_out.at[pl.ds(i * ne, ne), :],
            eout_sem.at[i % _EBUF])
        s.start()
        e_stores[i] = s
        nxt = i + _EBUF
        if nxt < _CE:
            e_stores[i].wait()
            c = pltpu.make_async_copy(
                e_hbm.at[pl.ds(nxt * ne, ne), :], ebufs[nxt % _EBUF],
                ein_sem.at[nxt % _EBUF])
            c.start()
            e_loads[nxt] = c

    for s in x_stores:
        s.wait()
    for i in range(_CE - _EBUF, _CE):
        e_stores[i].wait()


def kernel(x, x_lstm, encoded_z_gnss, edge_index, edge_attr):
    n_nodes, d_feat = x.shape
    n_edges, d_edge = edge_attr.shape
    x_out, e_out = pl.pallas_call(
        _copy_body,
        out_shape=(
            jax.ShapeDtypeStruct(x.shape, x.dtype),
            jax.ShapeDtypeStruct(edge_attr.shape, edge_attr.dtype),
        ),
        in_specs=[
            pl.BlockSpec(memory_space=pl.ANY),
            pl.BlockSpec(memory_space=pl.ANY),
        ],
        out_specs=(
            pl.BlockSpec(memory_space=pl.ANY),
            pl.BlockSpec(memory_space=pl.ANY),
        ),
        scratch_shapes=(
            [pltpu.MemorySpace.VMEM((n_nodes, d_feat), jnp.float32)]
            + [pltpu.MemorySpace.VMEM((n_edges // _CE, d_edge), jnp.float32)
               for _ in range(_EBUF)]
            + [pltpu.SemaphoreType.DMA((_CX,)),
               pltpu.SemaphoreType.DMA((_CX,)),
               pltpu.SemaphoreType.DMA((_EBUF,)),
               pltpu.SemaphoreType.DMA((_EBUF,))]
        ),
    )(x, edge_attr)
    return (x_out, e_out)
